# baseline (speedup 1.0000x reference)
import functools
import jax
import jax.numpy as jnp
from jax import lax
from jax.experimental import pallas as pl
from jax.experimental.pallas import tpu as pltpu
from jax.experimental.pallas import tpu_sc as plsc

_P = 4194304
_NW = 32
_SPAN = _P // _NW
_CH = 32768
_NCH = _SPAN // _CH
_L = 16

def _sc_body(a_hbm, b_hbm, out_hbm, a_v, zbuf, buf, sem_a, sem_in, sem_out):
    nc = 2
    wid = lax.axis_index("s") * nc + lax.axis_index("c")
    base = wid * _SPAN

    pltpu.make_async_copy(a_hbm, a_v, sem_a).start()

    def _z(k, _):
        zbuf[pl.ds(k * _L, _L)] = jnp.zeros((_L,), jnp.float32)
        return _

    lax.fori_loop(0, _CH // _L, _z, 0)
    pltpu.make_async_copy(a_hbm, a_v, sem_a).wait()

    for b in range(2):
        av = a_v[b]
        a_s = av[0]

        def in_cp(k, slot):
            return pltpu.make_async_copy(
                b_hbm.at[b, 0, pl.ds(base + k * _CH, _CH)],
                buf.at[slot],
                sem_in.at[slot],
            )

        def out_cp(k, slot):
            return pltpu.make_async_copy(
                buf.at[slot],
                out_hbm.at[b, 0, pl.ds(base + k * _CH, _CH)],
                sem_out.at[slot],
            )

        @pl.when(a_s == 0.0)
        def _():
            for k in range(_NCH):
                pltpu.make_async_copy(
                    zbuf,
                    out_hbm.at[b, 0, pl.ds(base + k * _CH, _CH)],
                    sem_out.at[0],
                ).start()
            for k in range(_NCH):
                pltpu.make_async_copy(
                    zbuf,
                    out_hbm.at[b, 0, pl.ds(base + k * _CH, _CH)],
                    sem_out.at[0],
                ).wait()

        @pl.when(a_s == 1.0)
        def _():
            in_cp(0, 0).start()
            for k in range(_NCH):
                if k + 1 < _NCH:
                    if k >= 1:
                        out_cp(k - 1, (k + 1) % 2).wait()
                    in_cp(k + 1, (k + 1) % 2).start()
                in_cp(k, k % 2).wait()
                out_cp(k, k % 2).start()
            for k in range(max(_NCH - 2, 0), _NCH):
                out_cp(k, k % 2).wait()

        @pl.when(jnp.logical_and(a_s != 0.0, a_s != 1.0))
        def _():
            for k in range(_NCH):
                in_cp(k, 0).start()
                in_cp(k, 0).wait()

                def _m(m, _):
                    v = buf[0, pl.ds(m * _L, _L)]
                    buf[0, pl.ds(m * _L, _L)] = v * av
                    return _

                lax.fori_loop(0, _CH // _L, _m, 0)
                out_cp(k, 0).start()
                out_cp(k, 0).wait()


def sc_kernel(B, A):
    a16 = jnp.broadcast_to(A.reshape(2, 1), (2, _L))
    run = pl.kernel(
        _sc_body,
        out_type=jax.ShapeDtypeStruct((2, 1, _P), jnp.float32),
        mesh=plsc.VectorSubcoreMesh(core_axis_name="c", subcore_axis_name="s"),
        scratch_types=[
            pltpu.VMEM((2, _L), jnp.float32),
            pltpu.VMEM((_CH,), jnp.float32),
            pltpu.VMEM((2, _CH), jnp.float32),
            pltpu.SemaphoreType.DMA,
            pltpu.SemaphoreType.DMA((2,)),
            pltpu.SemaphoreType.DMA((2,)),
        ],
    )
    return run(a16, B)


kernel = sc_kernel


# speedup vs baseline: 2.9376x; 2.9376x over previous
import jax
import jax.numpy as jnp
from jax.experimental import pallas as pl
from jax.experimental.pallas import tpu as pltpu

_P = 4194304
_CHUNK = 1 << 21
_NCHUNK = _P // _CHUNK
_TOTAL = 2 * _NCHUNK
_NBUF = 3


def _body(a_smem, b_any, out_vmem, inb, sems):
    bi = pl.program_id(0)
    j = pl.program_id(1)
    i = bi * _NCHUNK + j

    def in_copy(b_idx, j_idx, slot):
        return pltpu.make_async_copy(
            b_any.at[b_idx, pl.ds(0, 1), pl.ds(j_idx * _CHUNK, _CHUNK)],
            inb.at[slot],
            sems.at[slot],
        )

    @pl.when(i == 0)
    def _():
        for c in range(min(2, _TOTAL)):
            cb, cj = divmod(c, _NCHUNK)

            @pl.when(a_smem[cb] != 0.0)
            def _():
                in_copy(cb, cj, c % _NBUF).start()

    i2 = i + 2
    b2 = jnp.minimum(i2 // _NCHUNK, 1)
    j2 = i2 % _NCHUNK

    @pl.when(jnp.logical_and(i2 < _TOTAL, a_smem[b2] != 0.0))
    def _():
        in_copy(b2, j2, i2 % _NBUF).start()

    a = a_smem[bi]

    @pl.when(a != 0.0)
    def _():
        in_copy(bi, j, i % _NBUF).wait()
        out_vmem[0] = a * inb[i % _NBUF]

    @pl.when(a == 0.0)
    def _():
        out_vmem[0] = jnp.zeros((1, _CHUNK), jnp.float32)


def kernel(B, A):
    a2 = A.reshape(2)
    out = pl.pallas_call(
        _body,
        grid=(2, _NCHUNK),
        in_specs=[
            pl.BlockSpec(memory_space=pltpu.SMEM),
            pl.BlockSpec(memory_space=pl.ANY),
        ],
        out_specs=pl.BlockSpec((1, 1, _CHUNK), lambda b, j: (b, 0, j)),
        out_shape=jax.ShapeDtypeStruct((2, 1, _P), jnp.float32),
        scratch_shapes=[
            pltpu.VMEM((_NBUF, 1, _CHUNK), jnp.float32),
            pltpu.SemaphoreType.DMA((_NBUF,)),
        ],
    )(a2, B)
    return out
